# Initial kernel scaffold; baseline (speedup 1.0000x reference)
import jax
import jax.numpy as jnp
from jax.experimental import pallas as pl


def kernel(x, edge_index, Wn0, bn0, Ws0, Wn1, bn1, Ws1, Wn2, bn2, Ws2, Wfc, bfc):
    raise NotImplementedError("write your pallas kernel here")


# speedup vs baseline: 11.8861x; 11.8861x over previous
import functools

import jax
import jax.numpy as jnp
from jax import lax
from jax.experimental import pallas as pl
from jax.experimental.pallas import tpu as pltpu
from jax.experimental.pallas import tpu_sc as plsc

N = 10000
E = 320000
DIM = 32

NC = 2
NS = 16
NW = NC * NS

BATCH = 125
OPS_PER_TILE = (E // NW) // BATCH
ROWS_PER_TILE = 624

@functools.cache
def _make_segsum_sc():
    mesh = plsc.VectorSubcoreMesh(core_axis_name="c", subcore_axis_name="s",
                                  num_cores=NC, num_subcores=NS)
    return pl.kernel(
        _segsum_sc_body,
        out_type=jax.ShapeDtypeStruct((NC, N, DIM), jnp.float32),
        mesh=mesh,
        scratch_types=[
            pltpu.VMEM((OPS_PER_TILE, BATCH), jnp.int32),
            pltpu.VMEM((OPS_PER_TILE, BATCH), jnp.int32),
            pltpu.VMEM((BATCH, DIM), jnp.float32),
            pltpu.VMEM_SHARED((N, DIM), jnp.float32),
            pltpu.SemaphoreType.DMA,
        ],
        compiler_params=pltpu.CompilerParams(use_tc_tiling_on_sc=False),
    )


def _segsum_sc(p, src2d, dst2d, zeros):
    return _make_segsum_sc()(p, src2d, dst2d, zeros)


def _segsum_sc_body(p_hbm, src_hbm, dst_hbm, zeros_hbm, out_hbm,
                    src_idx, dst_idx, rows, agg, sem):
    c = lax.axis_index("c")
    s = lax.axis_index("s")

    row0 = s * ROWS_PER_TILE
    pltpu.sync_copy(zeros_hbm.at[pl.ds(row0, ROWS_PER_TILE)],
                    agg.at[pl.ds(row0, ROWS_PER_TILE)])

    @pl.when(s == NS - 1)
    def _zero_tail():
        pltpu.sync_copy(zeros_hbm.at[pl.ds(NS * ROWS_PER_TILE,
                                           N - NS * ROWS_PER_TILE)],
                        agg.at[pl.ds(NS * ROWS_PER_TILE,
                                     N - NS * ROWS_PER_TILE)])

    idx_row0 = (c * NS + s) * OPS_PER_TILE
    pltpu.sync_copy(src_hbm.at[pl.ds(idx_row0, OPS_PER_TILE)], src_idx)
    pltpu.sync_copy(dst_hbm.at[pl.ds(idx_row0, OPS_PER_TILE)], dst_idx)

    plsc.subcore_barrier()

    @pl.loop(0, OPS_PER_TILE)
    def _edge_chunk(j):
        pltpu.async_copy(p_hbm.at[src_idx.at[j]], rows, sem).wait()
        pltpu.sync_copy(rows, agg.at[dst_idx.at[j]], add=True)

    plsc.subcore_barrier()

    pltpu.sync_copy(agg.at[pl.ds(row0, ROWS_PER_TILE)],
                    out_hbm.at[c, pl.ds(row0, ROWS_PER_TILE)])

    @pl.when(s == NS - 1)
    def _out_tail():
        pltpu.sync_copy(agg.at[pl.ds(NS * ROWS_PER_TILE,
                                     N - NS * ROWS_PER_TILE)],
                        out_hbm.at[c, pl.ds(NS * ROWS_PER_TILE,
                                            N - NS * ROWS_PER_TILE)])


BN = 1000


def _mm0_body(x_ref, wn_ref, ws_ref, p_ref, s_ref):
    x = x_ref[...]
    p_ref[...] = jnp.dot(x, wn_ref[...], preferred_element_type=jnp.float32)
    s_ref[...] = jnp.dot(x, ws_ref[...], preferred_element_type=jnp.float32)


def _layer0_matmul(x, Wn0, Ws0):
    return pl.pallas_call(
        _mm0_body,
        grid=(N // BN,),
        in_specs=[
            pl.BlockSpec((BN, 128), lambda i: (i, 0)),
            pl.BlockSpec((128, DIM), lambda i: (0, 0)),
            pl.BlockSpec((128, DIM), lambda i: (0, 0)),
        ],
        out_specs=[
            pl.BlockSpec((BN, DIM), lambda i: (i, 0)),
            pl.BlockSpec((BN, DIM), lambda i: (i, 0)),
        ],
        out_shape=[
            jax.ShapeDtypeStruct((N, DIM), jnp.float32),
            jax.ShapeDtypeStruct((N, DIM), jnp.float32),
        ],
    )(x, Wn0, Ws0)


def _combine_body(agg_ref, sp_ref, bn_ref, wn_ref, ws_ref,
                  h_ref, p_ref, s_ref):
    h = jnp.maximum(agg_ref[0] + agg_ref[1] + bn_ref[...] + sp_ref[...], 0.0)
    h_ref[...] = h
    p_ref[...] = jnp.dot(h, wn_ref[...], preferred_element_type=jnp.float32)
    s_ref[...] = jnp.dot(h, ws_ref[...], preferred_element_type=jnp.float32)


def _combine_matmul(agg, s_prev, bn, Wn, Ws):
    return pl.pallas_call(
        _combine_body,
        grid=(N // BN,),
        in_specs=[
            pl.BlockSpec((NC, BN, DIM), lambda i: (0, i, 0)),
            pl.BlockSpec((BN, DIM), lambda i: (i, 0)),
            pl.BlockSpec((1, DIM), lambda i: (0, 0)),
            pl.BlockSpec((DIM, DIM), lambda i: (0, 0)),
            pl.BlockSpec((DIM, DIM), lambda i: (0, 0)),
        ],
        out_specs=[
            pl.BlockSpec((BN, DIM), lambda i: (i, 0)),
            pl.BlockSpec((BN, DIM), lambda i: (i, 0)),
            pl.BlockSpec((BN, DIM), lambda i: (i, 0)),
        ],
        out_shape=[
            jax.ShapeDtypeStruct((N, DIM), jnp.float32),
            jax.ShapeDtypeStruct((N, DIM), jnp.float32),
            jax.ShapeDtypeStruct((N, DIM), jnp.float32),
        ],
    )(agg, s_prev, bn.reshape(1, DIM), Wn, Ws)


def _final_body(x_ref, h1_ref, h2_ref, agg_ref, s2_ref, bn2_ref,
                wfc_ref, bfc_ref, out_ref):
    h3 = jnp.maximum(agg_ref[0] + agg_ref[1] + bn2_ref[...] + s2_ref[...], 0.0)
    t = jnp.dot(x_ref[...], wfc_ref[0:128, :],
                preferred_element_type=jnp.float32)
    t += jnp.dot(h1_ref[...], wfc_ref[128:160, :],
                 preferred_element_type=jnp.float32)
    t += jnp.dot(h2_ref[...], wfc_ref[160:192, :],
                 preferred_element_type=jnp.float32)
    t += jnp.dot(h3, wfc_ref[192:224, :],
                 preferred_element_type=jnp.float32)
    t += bfc_ref[...]
    m = jnp.max(t, axis=1, keepdims=True)
    lse = m + jnp.log(jnp.sum(jnp.exp(t - m), axis=1, keepdims=True))
    out_ref[...] = t - lse


def _final(x, h1, h2, agg2, s2, bn2, Wfc, bfc):
    n_classes = Wfc.shape[1]
    return pl.pallas_call(
        _final_body,
        grid=(N // BN,),
        in_specs=[
            pl.BlockSpec((BN, 128), lambda i: (i, 0)),
            pl.BlockSpec((BN, DIM), lambda i: (i, 0)),
            pl.BlockSpec((BN, DIM), lambda i: (i, 0)),
            pl.BlockSpec((NC, BN, DIM), lambda i: (0, i, 0)),
            pl.BlockSpec((BN, DIM), lambda i: (i, 0)),
            pl.BlockSpec((1, DIM), lambda i: (0, 0)),
            pl.BlockSpec((224, n_classes), lambda i: (0, 0)),
            pl.BlockSpec((1, n_classes), lambda i: (0, 0)),
        ],
        out_specs=pl.BlockSpec((BN, n_classes), lambda i: (i, 0)),
        out_shape=jax.ShapeDtypeStruct((N, n_classes), jnp.float32),
    )(x, h1, h2, agg2, s2, bn2.reshape(1, DIM), Wfc,
      bfc.reshape(1, n_classes))


def kernel(x, edge_index, Wn0, bn0, Ws0, Wn1, bn1, Ws1, Wn2, bn2, Ws2,
           Wfc, bfc):
    src2d = edge_index[0].reshape(E // BATCH, BATCH)
    dst2d = edge_index[1].reshape(E // BATCH, BATCH)
    zeros = jnp.zeros((N, DIM), jnp.float32)

    p0, s0 = _layer0_matmul(x, Wn0, Ws0)
    a0 = _segsum_sc(p0, src2d, dst2d, zeros)
    h1, p1, s1 = _combine_matmul(a0, s0, bn0, Wn1, Ws1)
    a1 = _segsum_sc(p1, src2d, dst2d, zeros)
    h2, p2, s2 = _combine_matmul(a1, s1, bn1, Wn2, Ws2)
    a2 = _segsum_sc(p2, src2d, dst2d, zeros)
    return _final(x, h1, h2, a2, s2, bn2, Wfc, bfc)


# speedup vs baseline: 16.4663x; 1.3853x over previous
import functools

import jax
import jax.numpy as jnp
from jax import lax
from jax.experimental import pallas as pl
from jax.experimental.pallas import tpu as pltpu
from jax.experimental.pallas import tpu_sc as plsc

N = 10000
E = 320000
DIM = 32

NC = 2
NS = 16
NW = NC * NS

BATCH = 125
OPS_PER_TILE = (E // NW) // BATCH
ROWS_PER_TILE = 624

@functools.cache
def _make_segsum_sc():
    mesh = plsc.VectorSubcoreMesh(core_axis_name="c", subcore_axis_name="s",
                                  num_cores=NC, num_subcores=NS)
    return pl.kernel(
        _segsum_sc_body,
        out_type=jax.ShapeDtypeStruct((NC, N, DIM), jnp.float32),
        mesh=mesh,
        scratch_types=[
            pltpu.VMEM((OPS_PER_TILE, BATCH), jnp.int32),
            pltpu.VMEM((OPS_PER_TILE, BATCH), jnp.int32),
            pltpu.VMEM((BATCH, DIM), jnp.float32),
            pltpu.VMEM((BATCH, DIM), jnp.float32),
            pltpu.VMEM_SHARED((N, DIM), jnp.float32),
            pltpu.SemaphoreType.DMA,
            pltpu.SemaphoreType.DMA,
        ],
        compiler_params=pltpu.CompilerParams(use_tc_tiling_on_sc=False),
    )


def _segsum_sc(p, src2d, dst2d, zeros):
    return _make_segsum_sc()(p, src2d, dst2d, zeros)


def _segsum_sc_body(p_hbm, src_hbm, dst_hbm, zeros_hbm, out_hbm,
                    src_idx, dst_idx, rows0, rows1, agg, sem0, sem1):
    c = lax.axis_index("c")
    s = lax.axis_index("s")

    row0 = s * ROWS_PER_TILE
    pltpu.sync_copy(zeros_hbm.at[pl.ds(row0, ROWS_PER_TILE)],
                    agg.at[pl.ds(row0, ROWS_PER_TILE)])

    @pl.when(s == NS - 1)
    def _zero_tail():
        pltpu.sync_copy(zeros_hbm.at[pl.ds(NS * ROWS_PER_TILE,
                                           N - NS * ROWS_PER_TILE)],
                        agg.at[pl.ds(NS * ROWS_PER_TILE,
                                     N - NS * ROWS_PER_TILE)])

    idx_row0 = (c * NS + s) * OPS_PER_TILE
    pltpu.sync_copy(src_hbm.at[pl.ds(idx_row0, OPS_PER_TILE)], src_idx)
    pltpu.sync_copy(dst_hbm.at[pl.ds(idx_row0, OPS_PER_TILE)], dst_idx)

    plsc.subcore_barrier()

    bufs = (rows0, rows1)
    sems = (sem0, sem1)
    pltpu.async_copy(p_hbm.at[src_idx.at[0]], rows0, sem0)
    pltpu.async_copy(p_hbm.at[src_idx.at[1]], rows1, sem1)

    @pl.loop(0, OPS_PER_TILE // 2)
    def _edge_pair(jj):
        for b in range(2):
            j = 2 * jj + b
            pltpu.make_async_copy(p_hbm.at[src_idx.at[j]], bufs[b],
                                  sems[b]).wait()
            pltpu.sync_copy(bufs[b], agg.at[dst_idx.at[j]], add=True)

            @pl.when(jj < OPS_PER_TILE // 2 - 1)
            def _next():
                pltpu.async_copy(p_hbm.at[src_idx.at[j + 2]], bufs[b],
                                 sems[b])

    plsc.subcore_barrier()

    pltpu.sync_copy(agg.at[pl.ds(row0, ROWS_PER_TILE)],
                    out_hbm.at[c, pl.ds(row0, ROWS_PER_TILE)])

    @pl.when(s == NS - 1)
    def _out_tail():
        pltpu.sync_copy(agg.at[pl.ds(NS * ROWS_PER_TILE,
                                     N - NS * ROWS_PER_TILE)],
                        out_hbm.at[c, pl.ds(NS * ROWS_PER_TILE,
                                            N - NS * ROWS_PER_TILE)])


BN = 1000


def _mm0_body(x_ref, wn_ref, ws_ref, p_ref, s_ref):
    x = x_ref[...]
    p_ref[...] = jnp.dot(x, wn_ref[...], preferred_element_type=jnp.float32)
    s_ref[...] = jnp.dot(x, ws_ref[...], preferred_element_type=jnp.float32)


def _layer0_matmul(x, Wn0, Ws0):
    return pl.pallas_call(
        _mm0_body,
        grid=(N // BN,),
        in_specs=[
            pl.BlockSpec((BN, 128), lambda i: (i, 0)),
            pl.BlockSpec((128, DIM), lambda i: (0, 0)),
            pl.BlockSpec((128, DIM), lambda i: (0, 0)),
        ],
        out_specs=[
            pl.BlockSpec((BN, DIM), lambda i: (i, 0)),
            pl.BlockSpec((BN, DIM), lambda i: (i, 0)),
        ],
        out_shape=[
            jax.ShapeDtypeStruct((N, DIM), jnp.float32),
            jax.ShapeDtypeStruct((N, DIM), jnp.float32),
        ],
    )(x, Wn0, Ws0)


def _combine_body(agg_ref, sp_ref, bn_ref, wn_ref, ws_ref,
                  h_ref, p_ref, s_ref):
    h = jnp.maximum(agg_ref[0] + agg_ref[1] + bn_ref[...] + sp_ref[...], 0.0)
    h_ref[...] = h
    p_ref[...] = jnp.dot(h, wn_ref[...], preferred_element_type=jnp.float32)
    s_ref[...] = jnp.dot(h, ws_ref[...], preferred_element_type=jnp.float32)


def _combine_matmul(agg, s_prev, bn, Wn, Ws):
    return pl.pallas_call(
        _combine_body,
        grid=(N // BN,),
        in_specs=[
            pl.BlockSpec((NC, BN, DIM), lambda i: (0, i, 0)),
            pl.BlockSpec((BN, DIM), lambda i: (i, 0)),
            pl.BlockSpec((1, DIM), lambda i: (0, 0)),
            pl.BlockSpec((DIM, DIM), lambda i: (0, 0)),
            pl.BlockSpec((DIM, DIM), lambda i: (0, 0)),
        ],
        out_specs=[
            pl.BlockSpec((BN, DIM), lambda i: (i, 0)),
            pl.BlockSpec((BN, DIM), lambda i: (i, 0)),
            pl.BlockSpec((BN, DIM), lambda i: (i, 0)),
        ],
        out_shape=[
            jax.ShapeDtypeStruct((N, DIM), jnp.float32),
            jax.ShapeDtypeStruct((N, DIM), jnp.float32),
            jax.ShapeDtypeStruct((N, DIM), jnp.float32),
        ],
    )(agg, s_prev, bn.reshape(1, DIM), Wn, Ws)


def _final_body(x_ref, h1_ref, h2_ref, agg_ref, s2_ref, bn2_ref,
                wfc_ref, bfc_ref, out_ref):
    h3 = jnp.maximum(agg_ref[0] + agg_ref[1] + bn2_ref[...] + s2_ref[...], 0.0)
    t = jnp.dot(x_ref[...], wfc_ref[0:128, :],
                preferred_element_type=jnp.float32)
    t += jnp.dot(h1_ref[...], wfc_ref[128:160, :],
                 preferred_element_type=jnp.float32)
    t += jnp.dot(h2_ref[...], wfc_ref[160:192, :],
                 preferred_element_type=jnp.float32)
    t += jnp.dot(h3, wfc_ref[192:224, :],
                 preferred_element_type=jnp.float32)
    t += bfc_ref[...]
    m = jnp.max(t, axis=1, keepdims=True)
    lse = m + jnp.log(jnp.sum(jnp.exp(t - m), axis=1, keepdims=True))
    out_ref[...] = t - lse


def _final(x, h1, h2, agg2, s2, bn2, Wfc, bfc):
    n_classes = Wfc.shape[1]
    return pl.pallas_call(
        _final_body,
        grid=(N // BN,),
        in_specs=[
            pl.BlockSpec((BN, 128), lambda i: (i, 0)),
            pl.BlockSpec((BN, DIM), lambda i: (i, 0)),
            pl.BlockSpec((BN, DIM), lambda i: (i, 0)),
            pl.BlockSpec((NC, BN, DIM), lambda i: (0, i, 0)),
            pl.BlockSpec((BN, DIM), lambda i: (i, 0)),
            pl.BlockSpec((1, DIM), lambda i: (0, 0)),
            pl.BlockSpec((224, n_classes), lambda i: (0, 0)),
            pl.BlockSpec((1, n_classes), lambda i: (0, 0)),
        ],
        out_specs=pl.BlockSpec((BN, n_classes), lambda i: (i, 0)),
        out_shape=jax.ShapeDtypeStruct((N, n_classes), jnp.float32),
    )(x, h1, h2, agg2, s2, bn2.reshape(1, DIM), Wfc,
      bfc.reshape(1, n_classes))


def kernel(x, edge_index, Wn0, bn0, Ws0, Wn1, bn1, Ws1, Wn2, bn2, Ws2,
           Wfc, bfc):
    src2d = edge_index[0].reshape(E // BATCH, BATCH)
    dst2d = edge_index[1].reshape(E // BATCH, BATCH)
    zeros = jnp.zeros((N, DIM), jnp.float32)

    p0, s0 = _layer0_matmul(x, Wn0, Ws0)
    a0 = _segsum_sc(p0, src2d, dst2d, zeros)
    h1, p1, s1 = _combine_matmul(a0, s0, bn0, Wn1, Ws1)
    a1 = _segsum_sc(p1, src2d, dst2d, zeros)
    h2, p2, s2 = _combine_matmul(a1, s1, bn1, Wn2, Ws2)
    a2 = _segsum_sc(p2, src2d, dst2d, zeros)
    return _final(x, h1, h2, a2, s2, bn2, Wfc, bfc)
